# Initial kernel scaffold; baseline (speedup 1.0000x reference)
import jax
import jax.numpy as jnp
from jax.experimental import pallas as pl


def kernel(x, edge_index, edge_weight, sigma, enc1_w, enc1_b, enc2_w, enc2_b, enc3_w, enc3_b, zl_w, zl_b, dec1_w, dec1_b, dec2_w, dec2_b, dec3_w, dec3_b, xbar_w, xbar_b, g1_w, g2_w, g3_w, g4_w, g5_w, cluster_layer):
    raise NotImplementedError("write your pallas kernel here")


# speedup vs baseline: 4.8385x; 4.8385x over previous
import functools

import jax
import jax.numpy as jnp
from jax import lax
from jax.experimental import pallas as pl
from jax.experimental.pallas import tpu as pltpu
from jax.experimental.pallas import tpu_sc as plsc

_NC = 2
_NS = 16
_L = 16

_F32 = jnp.float32


def _relu(v):
    return jnp.maximum(v, 0.0)


def _pad2(a, r, c):
    return jnp.pad(a, ((0, r - a.shape[0]), (0, c - a.shape[1])))


def _pad_row(a, c):
    return jnp.pad(a, ((0, c - a.shape[0]),)).reshape(1, c)


def _spmm_partial(feat, row, col, w, zrows):
    n, c = feat.shape
    e = row.shape[0]
    nw = _NC * _NS
    ew = e // nw
    b = 80
    nb = ew // b
    rps = (n // _NS) // 8 * 8
    rlast = n - (_NS - 1) * rps
    mesh = plsc.VectorSubcoreMesh(
        core_axis_name="c", subcore_axis_name="s",
        num_cores=_NC, num_subcores=_NS)

    @functools.partial(
        pl.kernel,
        out_type=jax.ShapeDtypeStruct((_NC, n, c), _F32),
        mesh=mesh,
        compiler_params=pltpu.CompilerParams(
            use_tc_tiling_on_sc=(c % 128 == 0)),
        scratch_types=[
            pltpu.VMEM((b,), jnp.int32),
            pltpu.VMEM((b,), jnp.int32),
            pltpu.VMEM((b,), _F32),
            pltpu.VMEM((b, c), _F32),
            pltpu.VMEM_SHARED((n, c), _F32),
            pltpu.SemaphoreType.DMA,
        ],
    )
    def spmm_k(feat_h, row_h, col_h, w_h, z_h, out_h, cidx, ridx, wv, rows,
               acc, sem):
        cid = lax.axis_index("c")
        sid = lax.axis_index("s")
        wid = sid * _NC + cid
        start = pl.multiple_of(sid * rps, 8)

        @pl.when(sid < _NS - 1)
        def _():
            pltpu.sync_copy(z_h.at[pl.ds(0, rps)], acc.at[pl.ds(start, rps)])

        @pl.when(sid == _NS - 1)
        def _():
            pltpu.sync_copy(z_h, acc.at[pl.ds(start, rlast)])

        plsc.subcore_barrier()
        base = wid * ew

        def batch(bi, carry):
            off = base + bi * b
            pltpu.sync_copy(col_h.at[pl.ds(off, b)], cidx)
            pltpu.sync_copy(row_h.at[pl.ds(off, b)], ridx)
            pltpu.sync_copy(w_h.at[pl.ds(off, b)], wv)
            pltpu.async_copy(feat_h.at[cidx], rows, sem).wait()

            def group(g, c2):
                wvec = wv[pl.ds(g * _L, _L)]
                for i in range(_L):
                    j = g * _L + i
                    wj = wvec[i]
                    for k in range(c // _L):
                        sl = pl.ds(k * _L, _L)
                        rows[j, sl] = rows[j, sl] * wj
                return c2

            lax.fori_loop(0, b // _L, group, 0)
            pltpu.sync_copy(rows, acc.at[ridx], add=True)
            return carry

        lax.fori_loop(0, nb, batch, 0)
        plsc.subcore_barrier()

        @pl.when(sid < _NS - 1)
        def _():
            pltpu.sync_copy(acc.at[pl.ds(start, rps)],
                            out_h.at[cid, pl.ds(start, rps)])

        @pl.when(sid == _NS - 1)
        def _():
            pltpu.sync_copy(acc.at[pl.ds(start, rlast)],
                            out_h.at[cid, pl.ds(start, rlast)])

    return spmm_k(feat, row, col, w, zrows)


def _row_spec(blk, cols):
    return pl.BlockSpec((blk, cols), lambda i: (i, 0))


def _part_spec(blk, cols):
    return pl.BlockSpec((2, blk, cols), lambda i: (0, i, 0))


def _full_spec(rows, cols):
    return pl.BlockSpec((rows, cols), lambda i: (0, 0))


def _smem_spec():
    return pl.BlockSpec(memory_space=pltpu.SMEM)


def _tc0(x, ws, cT, n, blk):
    grid = (n // blk,)

    def body(x_r, e1w, e1b, e2w, e2b, e3w, e3b, zlw, zlb, d1w, d1b, d2w, d2b,
             d3w, d3b, xbw, xbb, ct_r, t1_r, t2_r, t3_r, z_r, xbar_r, q_r):
        xb = x_r[...]
        t1 = _relu(xb @ e1w[...] + e1b[...])
        t2 = _relu(t1 @ e2w[...] + e2b[...])
        t3 = _relu(t2 @ e3w[...] + e3b[...])
        z = t3 @ zlw[...] + zlb[...]
        d1 = _relu(z @ d1w[...] + d1b[...])
        d2 = _relu(d1 @ d2w[...] + d2b[...])
        d3 = _relu(d2 @ d3w[...] + d3b[...])
        t1_r[...] = t1
        t2_r[...] = t2
        t3_r[...] = t3
        z_r[...] = z
        xbar_r[...] = d3 @ xbw[...] + xbb[...]
        ct = ct_r[...]
        zn = jnp.sum(z * z, axis=1, keepdims=True)
        cn = jnp.sum(ct * ct, axis=0, keepdims=True)
        dist = zn + cn - 2.0 * (z @ ct)
        q = 1.0 / (1.0 + dist)
        mask = lax.broadcasted_iota(jnp.int32, (1, 16), 1) < 10
        q = jnp.where(mask, q, 0.0)
        q = q / jnp.sum(q, axis=1, keepdims=True)
        q_r[...] = q[:, :10]

    return pl.pallas_call(
        body,
        grid=grid,
        in_specs=[
            _row_spec(blk, 128),
            _full_spec(128, 512), _full_spec(1, 512),
            _full_spec(512, 512), _full_spec(1, 512),
            _full_spec(512, 2048), _full_spec(1, 2048),
            _full_spec(2048, 16), _full_spec(1, 16),
            _full_spec(16, 2048), _full_spec(1, 2048),
            _full_spec(2048, 512), _full_spec(1, 512),
            _full_spec(512, 512), _full_spec(1, 512),
            _full_spec(512, 128), _full_spec(1, 128),
            _full_spec(16, 16),
        ],
        out_specs=[
            _row_spec(blk, 512), _row_spec(blk, 512), _row_spec(blk, 2048),
            _row_spec(blk, 16), _row_spec(blk, 128), _row_spec(blk, 10),
        ],
        out_shape=[
            jax.ShapeDtypeStruct((n, 512), _F32),
            jax.ShapeDtypeStruct((n, 512), _F32),
            jax.ShapeDtypeStruct((n, 2048), _F32),
            jax.ShapeDtypeStruct((n, 16), _F32),
            jax.ShapeDtypeStruct((n, 128), _F32),
            jax.ShapeDtypeStruct((n, 10), _F32),
        ],
    )(x, ws["e1w"], ws["e1b"], ws["e2w"], ws["e2b"], ws["e3w"], ws["e3b"],
      ws["zlw"], ws["zlb"], ws["d1w"], ws["d1b"], ws["d2w"], ws["d2b"],
      ws["d3w"], ws["d3b"], ws["xbw"], ws["xbb"], cT)


def _mix_phase(p_chunks, gw, tra, sig, n, blk, out_chunks):
    grid = (n // blk,)
    nin = len(p_chunks)
    kdim = 128 * nin
    ndim = 128 * out_chunks

    def body(*refs):
        sig_r = refs[0]
        p_rs = refs[1:1 + nin]
        gw_r = refs[1 + nin]
        tra_r = refs[2 + nin]
        out_rs = refs[3 + nin:]
        ps = jnp.concatenate([r[0] + r[1] for r in p_rs], axis=1)
        h = _relu(ps @ gw_r[...])
        s = sig_r[0]
        m = (1.0 - s) * h + s * tra_r[...]
        for k, o_r in enumerate(out_rs):
            o_r[...] = m[:, 128 * k:128 * (k + 1)]

    return pl.pallas_call(
        body,
        grid=grid,
        in_specs=[_smem_spec()]
        + [_part_spec(blk, 128) for _ in range(nin)]
        + [_full_spec(kdim, ndim), _row_spec(blk, ndim)],
        out_specs=[_row_spec(blk, 128) for _ in range(out_chunks)],
        out_shape=[jax.ShapeDtypeStruct((n, 128), _F32)
                   for _ in range(out_chunks)],
    )(sig, *p_chunks, gw, tra)


def _phase3(p_chunks, g3w, tra3, g4w, sig, n, blk):
    grid = (n // blk,)

    def body(sig_r, p0, p1, p2, p3, g3_r, tra_r, g4_r, t4_r):
        ps = jnp.concatenate([r[0] + r[1] for r in (p0, p1, p2, p3)], axis=1)
        h = _relu(ps @ g3_r[...])
        s = sig_r[0]
        m = (1.0 - s) * h + s * tra_r[...]
        t4_r[...] = m @ g4_r[...]

    return pl.pallas_call(
        body,
        grid=grid,
        in_specs=[_smem_spec()]
        + [_part_spec(blk, 128) for _ in range(4)]
        + [_full_spec(512, 2048), _row_spec(blk, 2048), _full_spec(2048, 16)],
        out_specs=[_row_spec(blk, 16)],
        out_shape=[jax.ShapeDtypeStruct((n, 16), _F32)],
    )(sig, *p_chunks, g3w, tra3, g4w)


def _phase4(p4, zp, g5w, sig, n, blk):
    grid = (n // blk,)

    def body(sig_r, p_r, z_r, g5_r, t5_r):
        h4 = _relu(p_r[0] + p_r[1])
        s = sig_r[0]
        m = (1.0 - s) * h4 + s * z_r[...]
        t5_r[...] = m @ g5_r[...]

    return pl.pallas_call(
        body,
        grid=grid,
        in_specs=[_smem_spec(), _part_spec(blk, 16), _row_spec(blk, 16),
                  _full_spec(16, 16)],
        out_specs=[_row_spec(blk, 16)],
        out_shape=[jax.ShapeDtypeStruct((n, 16), _F32)],
    )(sig, p4, zp, g5w)


def _phase5(p5, n, blk):
    grid = (n // blk,)

    def body(p_r, h_r, pred_r):
        h = p_r[0] + p_r[1]
        mask = lax.broadcasted_iota(jnp.int32, (1, 16), 1) < 10
        logits = jnp.where(mask, h, -1e30)
        m = jnp.max(logits, axis=1, keepdims=True)
        ex = jnp.exp(logits - m)
        sm = ex / jnp.sum(ex, axis=1, keepdims=True)
        h_r[...] = h[:, :10]
        pred_r[...] = sm[:, :10]

    return pl.pallas_call(
        body,
        grid=grid,
        in_specs=[_part_spec(blk, 16)],
        out_specs=[_row_spec(blk, 10), _row_spec(blk, 10)],
        out_shape=[jax.ShapeDtypeStruct((n, 10), _F32),
                   jax.ShapeDtypeStruct((n, 10), _F32)],
    )(p5)


def kernel(x, edge_index, edge_weight, sigma,
           enc1_w, enc1_b, enc2_w, enc2_b, enc3_w, enc3_b, zl_w, zl_b,
           dec1_w, dec1_b, dec2_w, dec2_b, dec3_w, dec3_b, xbar_w, xbar_b,
           g1_w, g2_w, g3_w, g4_w, g5_w, cluster_layer):
    n = x.shape[0]
    ws = {
        "e1w": _pad2(enc1_w, 128, 512), "e1b": _pad_row(enc1_b, 512),
        "e2w": _pad2(enc2_w, 512, 512), "e2b": _pad_row(enc2_b, 512),
        "e3w": _pad2(enc3_w, 512, 2048), "e3b": _pad_row(enc3_b, 2048),
        "zlw": _pad2(zl_w, 2048, 16), "zlb": _pad_row(zl_b, 16),
        "d1w": _pad2(dec1_w, 16, 2048), "d1b": _pad_row(dec1_b, 2048),
        "d2w": _pad2(dec2_w, 2048, 512), "d2b": _pad_row(dec2_b, 512),
        "d3w": _pad2(dec3_w, 512, 512), "d3b": _pad_row(dec3_b, 512),
        "xbw": _pad2(xbar_w, 512, 128), "xbb": xbar_b.reshape(1, 128),
    }
    g1w = _pad2(g1_w, 128, 512)
    g2w = _pad2(g2_w, 512, 512)
    g3w = _pad2(g3_w, 512, 2048)
    g4w = _pad2(g4_w, 2048, 16)
    g5w = _pad2(g5_w, 16, 16)
    cT = _pad2(cluster_layer, 16, 16).T

    row = edge_index[0]
    col = edge_index[1]
    w = edge_weight
    sig = sigma.reshape(1).astype(_F32)
    rlast = n - (_NS - 1) * ((n // _NS) // 8 * 8)
    z128 = jnp.zeros((rlast, 128), _F32)
    z16 = jnp.zeros((rlast, 16), _F32)

    tra1, tra2, tra3, zp, x_bar, q = _tc0(x, ws, cT, n, blk=1000)
    px = _spmm_partial(x, row, col, w, z128)

    m2 = _mix_phase([px], g1w, tra1, sig, n, 1000, out_chunks=4)
    p2 = [_spmm_partial(m2[k], row, col, w, z128) for k in range(4)]

    m3 = _mix_phase(p2, g2w, tra2, sig, n, 1000, out_chunks=4)
    p3 = [_spmm_partial(m3[k], row, col, w, z128) for k in range(4)]

    (t4,) = _phase3(p3, g3w, tra3, g4w, sig, n, 1000)
    p4 = _spmm_partial(t4, row, col, w, z16)

    (t5,) = _phase4(p4, zp, g5w, sig, n, 2000)
    p5 = _spmm_partial(t5, row, col, w, z16)

    h, predict = _phase5(p5, n, 2000)

    z = zp[:, :10]
    return (x_bar, q, predict, z, h)
